# Initial kernel scaffold; baseline (speedup 1.0000x reference)
import jax
import jax.numpy as jnp
from jax.experimental import pallas as pl


def kernel(token, edge_index, voq_W0, voq_b0, voq_W1, voq_b1, voq_W2, voq_b2, voq_W3, voq_b3, cnh_W0, cnh_b0, cnh_W1, cnh_b1, cnh_W2, cnh_b2, cnh_W3, cnh_b3):
    raise NotImplementedError("write your pallas kernel here")


# speedup vs baseline: 3.7363x; 3.7363x over previous
import functools

import jax
import jax.numpy as jnp
from jax import lax
from jax.experimental import pallas as pl
from jax.experimental.pallas import tpu as pltpu
from jax.experimental.pallas import tpu_sc as plsc

_N = 50000
_E = 800000
_LAT = 5
_HID = 50
_D = 16

_NC = 2
_NS = 16
_NW = _NC * _NS
_EPW = _E // _NW
_CH = 1000
_NCH = _EPW // _CH

_BE = 4000
_W = 128


def _sc_gather(table, src, tgt):
    mesh = plsc.VectorSubcoreMesh(core_axis_name="c", subcore_axis_name="s")
    out_type = (
        jax.ShapeDtypeStruct((_E, _D), jnp.float32),
        jax.ShapeDtypeStruct((_E, _D), jnp.float32),
    )

    @functools.partial(
        pl.kernel,
        out_type=out_type,
        mesh=mesh,
        compiler_params=pltpu.CompilerParams(use_tc_tiling_on_sc=False),
        scratch_types=[
            pltpu.VMEM((_CH,), jnp.int32),
            pltpu.VMEM((_CH,), jnp.int32),
            pltpu.VMEM((_CH, _D), jnp.float32),
            pltpu.VMEM((_CH, _D), jnp.float32),
            pltpu.SemaphoreType.DMA,
            pltpu.SemaphoreType.DMA,
        ],
    )
    def k(table_h, src_h, tgt_h, xs_h, xt_h, si, ti, sr, tr, sem_s, sem_t):
        wid = lax.axis_index("s") * _NC + lax.axis_index("c")

        def body(i, carry):
            base = wid * _EPW + i * _CH
            pltpu.sync_copy(src_h.at[pl.ds(base, _CH)], si)
            pltpu.sync_copy(tgt_h.at[pl.ds(base, _CH)], ti)
            cs = pltpu.async_copy(table_h.at[si], sr, sem_s)
            ct = pltpu.async_copy(table_h.at[ti], tr, sem_t)
            cs.wait()
            ct.wait()
            pltpu.sync_copy(sr, xs_h.at[pl.ds(base, _CH)])
            pltpu.sync_copy(tr, xt_h.at[pl.ds(base, _CH)])
            return carry

        lax.fori_loop(0, _NCH, body, 0)

    return k(table, src, tgt)


def _gelu(x):
    return 0.5 * x * (1.0 + lax.erf(x * 0.7071067811865476))


def _tc_body(xs_ref, xt_ref, ws_ref, wt_ref, b0_ref, w1_ref, b1_ref,
             w2_ref, b2_ref, w3_ref, b3_ref, o1_ref, o2_ref):
    f32 = jnp.float32
    h = jnp.dot(xs_ref[...], ws_ref[...], preferred_element_type=f32)
    h = h + jnp.dot(xt_ref[...], wt_ref[...], preferred_element_type=f32)
    h = _gelu(h + b0_ref[...])
    h = _gelu(jnp.dot(h, w1_ref[...], preferred_element_type=f32) + b1_ref[...])
    h = _gelu(jnp.dot(h, w2_ref[...], preferred_element_type=f32) + b2_ref[...])
    z = jnp.dot(h, w3_ref[...], preferred_element_type=f32) + b3_ref[...]
    o = 1.0 / (1.0 + jnp.exp(-z))
    o1_ref[...] = o[:, 0:1]
    o2_ref[...] = o[:, 1:2]


def _tc_mlp(xs, xt, ws, wt, b0, w1, b1, w2, b2, w3, b3):
    grid = (_E // _BE,)
    edge_spec = pl.BlockSpec((_BE, _D), lambda i: (i, 0))
    full = lambda shape: pl.BlockSpec(shape, lambda i: (0, 0))
    return pl.pallas_call(
        _tc_body,
        grid=grid,
        in_specs=[
            edge_spec,
            edge_spec,
            full((_D, _W)),
            full((_D, _W)),
            full((1, _W)),
            full((_W, _W)),
            full((1, _W)),
            full((_W, _W)),
            full((1, _W)),
            full((_W, 8)),
            full((1, 8)),
        ],
        out_specs=[
            pl.BlockSpec((_BE, 1), lambda i: (i, 0)),
            pl.BlockSpec((_BE, 1), lambda i: (i, 0)),
        ],
        out_shape=[
            jax.ShapeDtypeStruct((_E, 1), jnp.float32),
            jax.ShapeDtypeStruct((_E, 1), jnp.float32),
        ],
    )(xs, xt, ws, wt, b0, w1, b1, w2, b2, w3, b3)


def kernel(token, edge_index,
           voq_W0, voq_b0, voq_W1, voq_b1, voq_W2, voq_b2, voq_W3, voq_b3,
           cnh_W0, cnh_b0, cnh_W1, cnh_b1, cnh_W2, cnh_b2, cnh_W3, cnh_b3):
    f32 = jnp.float32
    table = jnp.zeros((_N, _D), f32).at[:, :_LAT].set(token)
    src = edge_index[0]
    tgt = edge_index[1]
    xs, xt = _sc_gather(table, src, tgt)

    ws = (jnp.zeros((_D, _W), f32)
          .at[:_LAT, 0:_HID].set(voq_W0[:_LAT])
          .at[:_LAT, _HID:2 * _HID].set(cnh_W0[:_LAT]))
    wt = (jnp.zeros((_D, _W), f32)
          .at[:_LAT, 0:_HID].set(voq_W0[_LAT:])
          .at[:_LAT, _HID:2 * _HID].set(cnh_W0[_LAT:]))
    b0 = (jnp.zeros((1, _W), f32)
          .at[0, 0:_HID].set(voq_b0)
          .at[0, _HID:2 * _HID].set(cnh_b0))
    w1 = (jnp.zeros((_W, _W), f32)
          .at[0:_HID, 0:_HID].set(voq_W1)
          .at[_HID:2 * _HID, _HID:2 * _HID].set(cnh_W1))
    b1 = (jnp.zeros((1, _W), f32)
          .at[0, 0:_HID].set(voq_b1)
          .at[0, _HID:2 * _HID].set(cnh_b1))
    w2 = (jnp.zeros((_W, _W), f32)
          .at[0:_HID, 0:_HID].set(voq_W2)
          .at[_HID:2 * _HID, _HID:2 * _HID].set(cnh_W2))
    b2 = (jnp.zeros((1, _W), f32)
          .at[0, 0:_HID].set(voq_b2)
          .at[0, _HID:2 * _HID].set(cnh_b2))
    w3 = (jnp.zeros((_W, 8), f32)
          .at[0:_HID, 0].set(voq_W3[:, 0])
          .at[_HID:2 * _HID, 1].set(cnh_W3[:, 0]))
    b3 = (jnp.zeros((1, 8), f32)
          .at[0, 0].set(voq_b3[0])
          .at[0, 1].set(cnh_b3[0]))

    voq, cnh = _tc_mlp(xs, xt, ws, wt, b0, w1, b1, w2, b2, w3, b3)
    return (voq, cnh)


# speedup vs baseline: 3.7795x; 1.0115x over previous
import functools

import jax
import jax.numpy as jnp
from jax import lax
from jax.experimental import pallas as pl
from jax.experimental.pallas import tpu as pltpu
from jax.experimental.pallas import tpu_sc as plsc

_N = 50000
_E = 800000
_LAT = 5
_HID = 50
_D = 8

_NC = 2
_NS = 16
_NW = _NC * _NS
_EPW = _E // _NW
_CH = 1000
_NCH = _EPW // _CH

_BE = 4000
_W = 128


def _sc_gather(table, src, tgt):
    mesh = plsc.VectorSubcoreMesh(core_axis_name="c", subcore_axis_name="s")
    out_type = (
        jax.ShapeDtypeStruct((_E, _D), jnp.float32),
        jax.ShapeDtypeStruct((_E, _D), jnp.float32),
    )

    @functools.partial(
        pl.kernel,
        out_type=out_type,
        mesh=mesh,
        compiler_params=pltpu.CompilerParams(use_tc_tiling_on_sc=False),
        scratch_types=[
            pltpu.VMEM((_CH,), jnp.int32),
            pltpu.VMEM((_CH,), jnp.int32),
            pltpu.VMEM((_CH,), jnp.int32),
            pltpu.VMEM((_CH,), jnp.int32),
            pltpu.VMEM((_CH, _D), jnp.float32),
            pltpu.VMEM((_CH, _D), jnp.float32),
            pltpu.VMEM((_CH, _D), jnp.float32),
            pltpu.VMEM((_CH, _D), jnp.float32),
            pltpu.SemaphoreType.DMA((2,)),
            pltpu.SemaphoreType.DMA((2,)),
            pltpu.SemaphoreType.DMA((2,)),
            pltpu.SemaphoreType.DMA((2,)),
            pltpu.SemaphoreType.DMA((2,)),
            pltpu.SemaphoreType.DMA((2,)),
        ],
    )
    def k(table_h, src_h, tgt_h, xs_h, xt_h,
          si0, si1, ti0, ti1, sr0, sr1, tr0, tr1,
          sidx_sem, tidx_sem, srow_sem, trow_sem, sout_sem, tout_sem):
        wid = lax.axis_index("s") * _NC + lax.axis_index("c")
        si = (si0, si1)
        ti = (ti0, ti1)
        sr = (sr0, sr1)
        tr = (tr0, tr1)

        def base_of(i):
            return wid * _EPW + i * _CH

        def start_idx(b, i):
            base = base_of(i)
            pltpu.async_copy(src_h.at[pl.ds(base, _CH)], si[b], sidx_sem.at[b])
            pltpu.async_copy(tgt_h.at[pl.ds(base, _CH)], ti[b], tidx_sem.at[b])

        def wait_idx(b):
            pltpu.make_async_copy(src_h.at[pl.ds(0, _CH)], si[b], sidx_sem.at[b]).wait()
            pltpu.make_async_copy(tgt_h.at[pl.ds(0, _CH)], ti[b], tidx_sem.at[b]).wait()

        def start_gather(b):
            pltpu.async_copy(table_h.at[si[b]], sr[b], srow_sem.at[b])
            pltpu.async_copy(table_h.at[ti[b]], tr[b], trow_sem.at[b])

        def wait_gather(b):
            pltpu.make_async_copy(table_h.at[si[b]], sr[b], srow_sem.at[b]).wait()
            pltpu.make_async_copy(table_h.at[ti[b]], tr[b], trow_sem.at[b]).wait()

        def start_out(b, i):
            base = base_of(i)
            pltpu.async_copy(sr[b], xs_h.at[pl.ds(base, _CH)], sout_sem.at[b])
            pltpu.async_copy(tr[b], xt_h.at[pl.ds(base, _CH)], tout_sem.at[b])

        def wait_out(b):
            pltpu.make_async_copy(sr[b], xs_h.at[pl.ds(0, _CH)], sout_sem.at[b]).wait()
            pltpu.make_async_copy(tr[b], xt_h.at[pl.ds(0, _CH)], tout_sem.at[b]).wait()

        def chunk_pair(g, carry):
            i0 = 2 * g
            i1 = i0 + 1
            start_idx(0, i0)
            start_idx(1, i1)
            wait_idx(0)

            @pl.when(g > 0)
            def _():
                wait_out(0)

            start_gather(0)
            wait_idx(1)

            @pl.when(g > 0)
            def _():
                wait_out(1)

            start_gather(1)
            wait_gather(0)
            start_out(0, i0)
            wait_gather(1)
            start_out(1, i1)
            return carry

        lax.fori_loop(0, _NCH // 2, chunk_pair, 0)

        i_t = _NCH - 1
        start_idx(0, i_t)
        wait_out(0)
        wait_idx(0)
        start_gather(0)
        wait_gather(0)
        start_out(0, i_t)
        wait_out(0)
        wait_out(1)

    return k(table, src, tgt)


def _gelu(x):
    return 0.5 * x * (1.0 + lax.erf(x * 0.7071067811865476))


def _tc_body(xs_ref, xt_ref, ws_ref, wt_ref, b0_ref, w1_ref, b1_ref,
             w2_ref, b2_ref, w3_ref, b3_ref, o1_ref, o2_ref):
    f32 = jnp.float32
    h = jnp.dot(xs_ref[...], ws_ref[...], preferred_element_type=f32)
    h = h + jnp.dot(xt_ref[...], wt_ref[...], preferred_element_type=f32)
    h = _gelu(h + b0_ref[...])
    h = _gelu(jnp.dot(h, w1_ref[...], preferred_element_type=f32) + b1_ref[...])
    h = _gelu(jnp.dot(h, w2_ref[...], preferred_element_type=f32) + b2_ref[...])
    z = jnp.dot(h, w3_ref[...], preferred_element_type=f32) + b3_ref[...]
    o = 1.0 / (1.0 + jnp.exp(-z))
    o1_ref[...] = o[:, 0:1]
    o2_ref[...] = o[:, 1:2]


def _tc_mlp(xs, xt, ws, wt, b0, w1, b1, w2, b2, w3, b3):
    grid = (_E // _BE,)
    edge_spec = pl.BlockSpec((_BE, _D), lambda i: (i, 0))
    full = lambda shape: pl.BlockSpec(shape, lambda i: (0, 0))
    return pl.pallas_call(
        _tc_body,
        grid=grid,
        in_specs=[
            edge_spec,
            edge_spec,
            full((_D, _W)),
            full((_D, _W)),
            full((1, _W)),
            full((_W, _W)),
            full((1, _W)),
            full((_W, _W)),
            full((1, _W)),
            full((_W, 8)),
            full((1, 8)),
        ],
        out_specs=[
            pl.BlockSpec((_BE, 1), lambda i: (i, 0)),
            pl.BlockSpec((_BE, 1), lambda i: (i, 0)),
        ],
        out_shape=[
            jax.ShapeDtypeStruct((_E, 1), jnp.float32),
            jax.ShapeDtypeStruct((_E, 1), jnp.float32),
        ],
    )(xs, xt, ws, wt, b0, w1, b1, w2, b2, w3, b3)


def kernel(token, edge_index,
           voq_W0, voq_b0, voq_W1, voq_b1, voq_W2, voq_b2, voq_W3, voq_b3,
           cnh_W0, cnh_b0, cnh_W1, cnh_b1, cnh_W2, cnh_b2, cnh_W3, cnh_b3):
    f32 = jnp.float32
    table = jnp.zeros((_N, _D), f32).at[:, :_LAT].set(token)
    src = edge_index[0]
    tgt = edge_index[1]
    xs, xt = _sc_gather(table, src, tgt)

    ws = (jnp.zeros((_D, _W), f32)
          .at[:_LAT, 0:_HID].set(voq_W0[:_LAT])
          .at[:_LAT, _HID:2 * _HID].set(cnh_W0[:_LAT]))
    wt = (jnp.zeros((_D, _W), f32)
          .at[:_LAT, 0:_HID].set(voq_W0[_LAT:])
          .at[:_LAT, _HID:2 * _HID].set(cnh_W0[_LAT:]))
    b0 = (jnp.zeros((1, _W), f32)
          .at[0, 0:_HID].set(voq_b0)
          .at[0, _HID:2 * _HID].set(cnh_b0))
    w1 = (jnp.zeros((_W, _W), f32)
          .at[0:_HID, 0:_HID].set(voq_W1)
          .at[_HID:2 * _HID, _HID:2 * _HID].set(cnh_W1))
    b1 = (jnp.zeros((1, _W), f32)
          .at[0, 0:_HID].set(voq_b1)
          .at[0, _HID:2 * _HID].set(cnh_b1))
    w2 = (jnp.zeros((_W, _W), f32)
          .at[0:_HID, 0:_HID].set(voq_W2)
          .at[_HID:2 * _HID, _HID:2 * _HID].set(cnh_W2))
    b2 = (jnp.zeros((1, _W), f32)
          .at[0, 0:_HID].set(voq_b2)
          .at[0, _HID:2 * _HID].set(cnh_b2))
    w3 = (jnp.zeros((_W, 8), f32)
          .at[0:_HID, 0].set(voq_W3[:, 0])
          .at[_HID:2 * _HID, 1].set(cnh_W3[:, 0]))
    b3 = (jnp.zeros((1, 8), f32)
          .at[0, 0].set(voq_b3[0])
          .at[0, 1].set(cnh_b3[0]))

    voq, cnh = _tc_mlp(xs, xt, ws, wt, b0, w1, b1, w2, b2, w3, b3)
    return (voq, cnh)


# speedup vs baseline: 3.8765x; 1.0257x over previous
import functools

import jax
import jax.numpy as jnp
from jax import lax
from jax.experimental import pallas as pl
from jax.experimental.pallas import tpu as pltpu
from jax.experimental.pallas import tpu_sc as plsc

_N = 50000
_E = 800000
_LAT = 5
_HID = 50
_D = 8

_NC = 2
_NS = 16
_NW = _NC * _NS
_EPW = _E // _NW
_CH = 1000
_NCH = _EPW // _CH

_BE = 4000
_W = 128


def _sc_gather(table, src, tgt):
    mesh = plsc.VectorSubcoreMesh(core_axis_name="c", subcore_axis_name="s")
    out_type = (
        jax.ShapeDtypeStruct((_E, _D), jnp.float32),
        jax.ShapeDtypeStruct((_E, _D), jnp.float32),
    )

    @functools.partial(
        pl.kernel,
        out_type=out_type,
        mesh=mesh,
        compiler_params=pltpu.CompilerParams(use_tc_tiling_on_sc=False),
        scratch_types=[
            pltpu.VMEM((_CH,), jnp.int32),
            pltpu.VMEM((_CH,), jnp.int32),
            pltpu.VMEM((_CH,), jnp.int32),
            pltpu.VMEM((_CH,), jnp.int32),
            pltpu.VMEM((_CH, _D), jnp.float32),
            pltpu.VMEM((_CH, _D), jnp.float32),
            pltpu.VMEM((_CH, _D), jnp.float32),
            pltpu.VMEM((_CH, _D), jnp.float32),
            pltpu.SemaphoreType.DMA((2,)),
            pltpu.SemaphoreType.DMA((2,)),
            pltpu.SemaphoreType.DMA((2,)),
            pltpu.SemaphoreType.DMA((2,)),
            pltpu.SemaphoreType.DMA((2,)),
            pltpu.SemaphoreType.DMA((2,)),
            pltpu.VMEM_SHARED((_N, _D), jnp.float32),
        ],
    )
    def k(table_h, src_h, tgt_h, xs_h, xt_h,
          si0, si1, ti0, ti1, sr0, sr1, tr0, tr1,
          sidx_sem, tidx_sem, srow_sem, trow_sem, sout_sem, tout_sem,
          table_sp):
        sid = lax.axis_index("s")
        wid = sid * _NC + lax.axis_index("c")

        @pl.when(sid == 0)
        def _():
            pltpu.sync_copy(table_h, table_sp)

        plsc.subcore_barrier()
        si = (si0, si1)
        ti = (ti0, ti1)
        sr = (sr0, sr1)
        tr = (tr0, tr1)

        def base_of(i):
            return wid * _EPW + i * _CH

        def start_idx(b, i):
            base = base_of(i)
            pltpu.async_copy(src_h.at[pl.ds(base, _CH)], si[b], sidx_sem.at[b])
            pltpu.async_copy(tgt_h.at[pl.ds(base, _CH)], ti[b], tidx_sem.at[b])

        def wait_idx(b):
            pltpu.make_async_copy(src_h.at[pl.ds(0, _CH)], si[b], sidx_sem.at[b]).wait()
            pltpu.make_async_copy(tgt_h.at[pl.ds(0, _CH)], ti[b], tidx_sem.at[b]).wait()

        def start_gather(b):
            pltpu.async_copy(table_sp.at[si[b]], sr[b], srow_sem.at[b])
            pltpu.async_copy(table_sp.at[ti[b]], tr[b], trow_sem.at[b])

        def wait_gather(b):
            pltpu.make_async_copy(table_sp.at[si[b]], sr[b], srow_sem.at[b]).wait()
            pltpu.make_async_copy(table_sp.at[ti[b]], tr[b], trow_sem.at[b]).wait()

        def start_out(b, i):
            base = base_of(i)
            pltpu.async_copy(sr[b], xs_h.at[pl.ds(base, _CH)], sout_sem.at[b])
            pltpu.async_copy(tr[b], xt_h.at[pl.ds(base, _CH)], tout_sem.at[b])

        def wait_out(b):
            pltpu.make_async_copy(sr[b], xs_h.at[pl.ds(0, _CH)], sout_sem.at[b]).wait()
            pltpu.make_async_copy(tr[b], xt_h.at[pl.ds(0, _CH)], tout_sem.at[b]).wait()

        def chunk_pair(g, carry):
            i0 = 2 * g
            i1 = i0 + 1
            start_idx(0, i0)
            start_idx(1, i1)
            wait_idx(0)

            @pl.when(g > 0)
            def _():
                wait_out(0)

            start_gather(0)
            wait_idx(1)

            @pl.when(g > 0)
            def _():
                wait_out(1)

            start_gather(1)
            wait_gather(0)
            start_out(0, i0)
            wait_gather(1)
            start_out(1, i1)
            return carry

        lax.fori_loop(0, _NCH // 2, chunk_pair, 0)

        i_t = _NCH - 1
        start_idx(0, i_t)
        wait_out(0)
        wait_idx(0)
        start_gather(0)
        wait_gather(0)
        start_out(0, i_t)
        wait_out(0)
        wait_out(1)

    return k(table, src, tgt)


def _gelu(x):
    return 0.5 * x * (1.0 + lax.erf(x * 0.7071067811865476))


def _tc_body(xs_ref, xt_ref, ws_ref, wt_ref, b0_ref, w1_ref, b1_ref,
             w2_ref, b2_ref, w3_ref, b3_ref, o1_ref, o2_ref):
    f32 = jnp.float32
    h = jnp.dot(xs_ref[...], ws_ref[...], preferred_element_type=f32)
    h = h + jnp.dot(xt_ref[...], wt_ref[...], preferred_element_type=f32)
    h = _gelu(h + b0_ref[...])
    h = _gelu(jnp.dot(h, w1_ref[...], preferred_element_type=f32) + b1_ref[...])
    h = _gelu(jnp.dot(h, w2_ref[...], preferred_element_type=f32) + b2_ref[...])
    z = jnp.dot(h, w3_ref[...], preferred_element_type=f32) + b3_ref[...]
    o = 1.0 / (1.0 + jnp.exp(-z))
    o1_ref[...] = o[:, 0:1]
    o2_ref[...] = o[:, 1:2]


def _tc_mlp(xs, xt, ws, wt, b0, w1, b1, w2, b2, w3, b3):
    grid = (_E // _BE,)
    edge_spec = pl.BlockSpec((_BE, _D), lambda i: (i, 0))
    full = lambda shape: pl.BlockSpec(shape, lambda i: (0, 0))
    return pl.pallas_call(
        _tc_body,
        grid=grid,
        in_specs=[
            edge_spec,
            edge_spec,
            full((_D, _W)),
            full((_D, _W)),
            full((1, _W)),
            full((_W, _W)),
            full((1, _W)),
            full((_W, _W)),
            full((1, _W)),
            full((_W, 8)),
            full((1, 8)),
        ],
        out_specs=[
            pl.BlockSpec((_BE, 1), lambda i: (i, 0)),
            pl.BlockSpec((_BE, 1), lambda i: (i, 0)),
        ],
        out_shape=[
            jax.ShapeDtypeStruct((_E, 1), jnp.float32),
            jax.ShapeDtypeStruct((_E, 1), jnp.float32),
        ],
    )(xs, xt, ws, wt, b0, w1, b1, w2, b2, w3, b3)


def kernel(token, edge_index,
           voq_W0, voq_b0, voq_W1, voq_b1, voq_W2, voq_b2, voq_W3, voq_b3,
           cnh_W0, cnh_b0, cnh_W1, cnh_b1, cnh_W2, cnh_b2, cnh_W3, cnh_b3):
    f32 = jnp.float32
    table = jnp.zeros((_N, _D), f32).at[:, :_LAT].set(token)
    src = edge_index[0]
    tgt = edge_index[1]
    xs, xt = _sc_gather(table, src, tgt)

    ws = (jnp.zeros((_D, _W), f32)
          .at[:_LAT, 0:_HID].set(voq_W0[:_LAT])
          .at[:_LAT, _HID:2 * _HID].set(cnh_W0[:_LAT]))
    wt = (jnp.zeros((_D, _W), f32)
          .at[:_LAT, 0:_HID].set(voq_W0[_LAT:])
          .at[:_LAT, _HID:2 * _HID].set(cnh_W0[_LAT:]))
    b0 = (jnp.zeros((1, _W), f32)
          .at[0, 0:_HID].set(voq_b0)
          .at[0, _HID:2 * _HID].set(cnh_b0))
    w1 = (jnp.zeros((_W, _W), f32)
          .at[0:_HID, 0:_HID].set(voq_W1)
          .at[_HID:2 * _HID, _HID:2 * _HID].set(cnh_W1))
    b1 = (jnp.zeros((1, _W), f32)
          .at[0, 0:_HID].set(voq_b1)
          .at[0, _HID:2 * _HID].set(cnh_b1))
    w2 = (jnp.zeros((_W, _W), f32)
          .at[0:_HID, 0:_HID].set(voq_W2)
          .at[_HID:2 * _HID, _HID:2 * _HID].set(cnh_W2))
    b2 = (jnp.zeros((1, _W), f32)
          .at[0, 0:_HID].set(voq_b2)
          .at[0, _HID:2 * _HID].set(cnh_b2))
    w3 = (jnp.zeros((_W, 8), f32)
          .at[0:_HID, 0].set(voq_W3[:, 0])
          .at[_HID:2 * _HID, 1].set(cnh_W3[:, 0]))
    b3 = (jnp.zeros((1, 8), f32)
          .at[0, 0].set(voq_b3[0])
          .at[0, 1].set(cnh_b3[0]))

    voq, cnh = _tc_mlp(xs, xt, ws, wt, b0, w1, b1, w2, b2, w3, b3)
    return (voq, cnh)


# speedup vs baseline: 6.2962x; 1.6242x over previous
import functools

import jax
import jax.numpy as jnp
from jax import lax
from jax.experimental import pallas as pl
from jax.experimental.pallas import tpu as pltpu
from jax.experimental.pallas import tpu_sc as plsc

_N = 50000
_E = 800000
_LAT = 5
_HID = 50
_D = 8

_NC = 2
_NS = 16
_NW = _NC * _NS
_EPW = _E // _NW
_CH = 1000
_NCH = _EPW // _CH

_PK = 128 // _D
_XR = _E * _D // 128
_XBB = 200
_BE = _XBB * _PK
_NB = _XR // _XBB
_W = 128


def _sc_gather(table, src, tgt):
    mesh = plsc.VectorSubcoreMesh(core_axis_name="c", subcore_axis_name="s")
    out_type = (
        jax.ShapeDtypeStruct((_E, _D), jnp.float32),
        jax.ShapeDtypeStruct((_E, _D), jnp.float32),
    )

    @functools.partial(
        pl.kernel,
        out_type=out_type,
        mesh=mesh,
        compiler_params=pltpu.CompilerParams(use_tc_tiling_on_sc=False),
        scratch_types=[
            pltpu.VMEM((_CH,), jnp.int32),
            pltpu.VMEM((_CH,), jnp.int32),
            pltpu.VMEM((_CH,), jnp.int32),
            pltpu.VMEM((_CH,), jnp.int32),
            pltpu.VMEM((_CH, _D), jnp.float32),
            pltpu.VMEM((_CH, _D), jnp.float32),
            pltpu.VMEM((_CH, _D), jnp.float32),
            pltpu.VMEM((_CH, _D), jnp.float32),
            pltpu.SemaphoreType.DMA((2,)),
            pltpu.SemaphoreType.DMA((2,)),
            pltpu.SemaphoreType.DMA((2,)),
            pltpu.SemaphoreType.DMA((2,)),
            pltpu.SemaphoreType.DMA((2,)),
            pltpu.SemaphoreType.DMA((2,)),
            pltpu.VMEM_SHARED((_N, _D), jnp.float32),
        ],
    )
    def k(table_h, src_h, tgt_h, xs_h, xt_h,
          si0, si1, ti0, ti1, sr0, sr1, tr0, tr1,
          sidx_sem, tidx_sem, srow_sem, trow_sem, sout_sem, tout_sem,
          table_sp):
        sid = lax.axis_index("s")
        wid = sid * _NC + lax.axis_index("c")

        @pl.when(sid == 0)
        def _():
            pltpu.sync_copy(table_h, table_sp)

        plsc.subcore_barrier()
        si = (si0, si1)
        ti = (ti0, ti1)
        sr = (sr0, sr1)
        tr = (tr0, tr1)

        def base_of(i):
            return wid * _EPW + i * _CH

        def start_idx(b, i):
            base = base_of(i)
            pltpu.async_copy(src_h.at[pl.ds(base, _CH)], si[b], sidx_sem.at[b])
            pltpu.async_copy(tgt_h.at[pl.ds(base, _CH)], ti[b], tidx_sem.at[b])

        def wait_idx(b):
            pltpu.make_async_copy(src_h.at[pl.ds(0, _CH)], si[b], sidx_sem.at[b]).wait()
            pltpu.make_async_copy(tgt_h.at[pl.ds(0, _CH)], ti[b], tidx_sem.at[b]).wait()

        def start_gather(b):
            pltpu.async_copy(table_sp.at[si[b]], sr[b], srow_sem.at[b])
            pltpu.async_copy(table_sp.at[ti[b]], tr[b], trow_sem.at[b])

        def wait_gather(b):
            pltpu.make_async_copy(table_sp.at[si[b]], sr[b], srow_sem.at[b]).wait()
            pltpu.make_async_copy(table_sp.at[ti[b]], tr[b], trow_sem.at[b]).wait()

        def start_out(b, i):
            base = base_of(i)
            pltpu.async_copy(sr[b], xs_h.at[pl.ds(base, _CH)], sout_sem.at[b])
            pltpu.async_copy(tr[b], xt_h.at[pl.ds(base, _CH)], tout_sem.at[b])

        def wait_out(b):
            pltpu.make_async_copy(sr[b], xs_h.at[pl.ds(0, _CH)], sout_sem.at[b]).wait()
            pltpu.make_async_copy(tr[b], xt_h.at[pl.ds(0, _CH)], tout_sem.at[b]).wait()

        def chunk_pair(g, carry):
            i0 = 2 * g
            i1 = i0 + 1
            start_idx(0, i0)
            start_idx(1, i1)
            wait_idx(0)

            @pl.when(g > 0)
            def _():
                wait_out(0)

            start_gather(0)
            wait_idx(1)

            @pl.when(g > 0)
            def _():
                wait_out(1)

            start_gather(1)
            wait_gather(0)
            start_out(0, i0)
            wait_gather(1)
            start_out(1, i1)
            return carry

        lax.fori_loop(0, _NCH // 2, chunk_pair, 0)

        i_t = _NCH - 1
        start_idx(0, i_t)
        wait_out(0)
        wait_idx(0)
        start_gather(0)
        wait_gather(0)
        start_out(0, i_t)
        wait_out(0)
        wait_out(1)

    return k(table, src, tgt)


def _gelu(x):
    return 0.5 * x * (1.0 + lax.erf(x * 0.7071067811865476))


def _tc_body(xs_ref, xt_ref, ws_ref, wt_ref, b0_ref, w1_ref, b1_ref,
             w2_ref, b2_ref, w3_ref, b3_ref, o1_ref, o2_ref):
    f32 = jnp.float32
    ws = ws_ref[...]
    wt = wt_ref[...]
    parts = []
    for j in range(_PK):
        exs = xs_ref[:, j * _D:(j + 1) * _D]
        ext = xt_ref[:, j * _D:(j + 1) * _D]
        h = jnp.dot(exs, ws, preferred_element_type=f32)
        h = h + jnp.dot(ext, wt, preferred_element_type=f32)
        parts.append(h)
    h = jnp.concatenate(parts, axis=0)
    h = _gelu(h + b0_ref[...])
    h = _gelu(jnp.dot(h, w1_ref[...], preferred_element_type=f32) + b1_ref[...])
    h = _gelu(jnp.dot(h, w2_ref[...], preferred_element_type=f32) + b2_ref[...])
    z = jnp.dot(h, w3_ref[...], preferred_element_type=f32) + b3_ref[...]
    o = 1.0 / (1.0 + jnp.exp(-z))
    o1_ref[...] = o[:, 0:1]
    o2_ref[...] = o[:, 1:2]


def _tc_mlp(xs, xt, ws, wt, b0, w1, b1, w2, b2, w3, b3):
    grid = (_NB,)
    edge_spec = pl.BlockSpec((_XBB, 128), lambda i: (i, 0))
    full = lambda shape: pl.BlockSpec(shape, lambda i: (0, 0))
    return pl.pallas_call(
        _tc_body,
        grid=grid,
        in_specs=[
            edge_spec,
            edge_spec,
            full((_D, _W)),
            full((_D, _W)),
            full((1, _W)),
            full((_W, _W)),
            full((1, _W)),
            full((_W, _W)),
            full((1, _W)),
            full((_W, 8)),
            full((1, 8)),
        ],
        out_specs=[
            pl.BlockSpec((_BE, 1), lambda i: (i, 0)),
            pl.BlockSpec((_BE, 1), lambda i: (i, 0)),
        ],
        out_shape=[
            jax.ShapeDtypeStruct((_E, 1), jnp.float32),
            jax.ShapeDtypeStruct((_E, 1), jnp.float32),
        ],
    )(xs, xt, ws, wt, b0, w1, b1, w2, b2, w3, b3)


def _unshuffle(o):
    return o.reshape(_NB, _PK, _XBB).transpose(0, 2, 1).reshape(_E, 1)


def kernel(token, edge_index,
           voq_W0, voq_b0, voq_W1, voq_b1, voq_W2, voq_b2, voq_W3, voq_b3,
           cnh_W0, cnh_b0, cnh_W1, cnh_b1, cnh_W2, cnh_b2, cnh_W3, cnh_b3):
    f32 = jnp.float32
    table = jnp.zeros((_N, _D), f32).at[:, :_LAT].set(token)
    src = edge_index[0]
    tgt = edge_index[1]
    xs, xt = _sc_gather(table, src, tgt)
    xs = xs.reshape(_XR, 128)
    xt = xt.reshape(_XR, 128)

    ws = (jnp.zeros((_D, _W), f32)
          .at[:_LAT, 0:_HID].set(voq_W0[:_LAT])
          .at[:_LAT, _HID:2 * _HID].set(cnh_W0[:_LAT]))
    wt = (jnp.zeros((_D, _W), f32)
          .at[:_LAT, 0:_HID].set(voq_W0[_LAT:])
          .at[:_LAT, _HID:2 * _HID].set(cnh_W0[_LAT:]))
    b0 = (jnp.zeros((1, _W), f32)
          .at[0, 0:_HID].set(voq_b0)
          .at[0, _HID:2 * _HID].set(cnh_b0))
    w1 = (jnp.zeros((_W, _W), f32)
          .at[0:_HID, 0:_HID].set(voq_W1)
          .at[_HID:2 * _HID, _HID:2 * _HID].set(cnh_W1))
    b1 = (jnp.zeros((1, _W), f32)
          .at[0, 0:_HID].set(voq_b1)
          .at[0, _HID:2 * _HID].set(cnh_b1))
    w2 = (jnp.zeros((_W, _W), f32)
          .at[0:_HID, 0:_HID].set(voq_W2)
          .at[_HID:2 * _HID, _HID:2 * _HID].set(cnh_W2))
    b2 = (jnp.zeros((1, _W), f32)
          .at[0, 0:_HID].set(voq_b2)
          .at[0, _HID:2 * _HID].set(cnh_b2))
    w3 = (jnp.zeros((_W, 8), f32)
          .at[0:_HID, 0].set(voq_W3[:, 0])
          .at[_HID:2 * _HID, 1].set(cnh_W3[:, 0]))
    b3 = (jnp.zeros((1, 8), f32)
          .at[0, 0].set(voq_b3[0])
          .at[0, 1].set(cnh_b3[0]))

    voq_i, cnh_i = _tc_mlp(xs, xt, ws, wt, b0, w1, b1, w2, b2, w3, b3)
    return (_unshuffle(voq_i), _unshuffle(cnh_i))
